# baseline (speedup 1.0000x reference)
import jax
import jax.numpy as jnp
from jax.experimental import pallas as pl

_F = 22
_BLK = 1000


def _split_body(x_ref, *out_refs):
    x = x_ref[...]
    for j in range(_F):
        out_refs[j][...] = x[:, j:j + 1]


def kernel(inputs):
    n = inputs.shape[0]
    outs = pl.pallas_call(
        _split_body,
        grid=(n // _BLK,),
        in_specs=[pl.BlockSpec((_BLK, _F), lambda i: (i, 0))],
        out_specs=[pl.BlockSpec((_BLK, 1), lambda i: (i, 0))] * _F,
        out_shape=[jax.ShapeDtypeStruct((n, 1), jnp.float32)] * _F,
    )(inputs)
    return tuple(outs)


# speedup vs baseline: 7.5324x; 7.5324x over previous
import functools

import jax
import jax.numpy as jnp
from jax import lax
from jax.experimental import pallas as pl
from jax.experimental.pallas import tpu as pltpu
from jax.experimental.pallas import tpu_sc as plsc

_F = 22
_NW = 32
_R = 2000
_L = 16


def _make_sc_kernel(n):
    rows_per_w = n // _NW
    chunks = rows_per_w // _R
    mesh = plsc.VectorSubcoreMesh(core_axis_name="c", subcore_axis_name="s")

    @functools.partial(
        pl.kernel,
        mesh=mesh,
        out_type=[jax.ShapeDtypeStruct((n,), jnp.float32)] * _F,
        scratch_types=[
            pltpu.VMEM((_R * _F,), jnp.float32),
            pltpu.VMEM((_F, _R), jnp.float32),
        ],
        compiler_params=pltpu.CompilerParams(
            needs_layout_passes=False, use_tc_tiling_on_sc=False
        ),
    )
    def sc_split(in_hbm, *rest):
        outs = rest[:_F]
        in_v = rest[_F]
        out_v = rest[_F + 1]
        wid = lax.axis_index("s") * 2 + lax.axis_index("c")
        base_w = wid * rows_per_w
        iota = lax.iota(jnp.int32, _L)
        iota_f = iota * _F

        def chunk_body(c, carry):
            row0 = base_w + c * _R
            pltpu.sync_copy(in_hbm.at[pl.ds(row0 * _F, _R * _F)], in_v)

            def r_body(r, carry2):
                base_vec = iota_f + r * (_L * _F)
                for j in range(_F):
                    idx = base_vec + j
                    vals = plsc.load_gather(in_v, [idx])
                    out_v[j, pl.ds(r * _L, _L)] = vals
                return carry2

            lax.fori_loop(0, _R // _L, r_body, 0, unroll=2)
            for j in range(_F):
                pltpu.sync_copy(out_v.at[j], outs[j].at[pl.ds(row0, _R)])
            return carry

        lax.fori_loop(0, chunks, chunk_body, 0)

    return sc_split


def kernel(inputs):
    n = inputs.shape[0]
    flat = inputs.reshape(-1)
    outs = _make_sc_kernel(n)(flat)
    return tuple(o.reshape(n, 1) for o in outs)


# speedup vs baseline: 8.0380x; 1.0671x over previous
import functools

import jax
import jax.numpy as jnp
from jax import lax
from jax.experimental import pallas as pl
from jax.experimental.pallas import tpu as pltpu
from jax.experimental.pallas import tpu_sc as plsc

_F = 22
_NW = 32
_R = 1280
_L = 16


def _make_sc_kernel(n):
    total_chunks = n // _R
    ch_base = total_chunks // _NW
    extra = total_chunks - ch_base * _NW
    mesh = plsc.VectorSubcoreMesh(core_axis_name="c", subcore_axis_name="s")

    @functools.partial(
        pl.kernel,
        mesh=mesh,
        out_type=[jax.ShapeDtypeStruct((n,), jnp.float32)] * _F,
        scratch_types=[
            pltpu.VMEM((_R * _F,), jnp.float32),
            pltpu.VMEM((_R * _F,), jnp.float32),
            pltpu.VMEM((_F, _R), jnp.float32),
            pltpu.VMEM((_F, _R), jnp.float32),
            pltpu.SemaphoreType.DMA,
            pltpu.SemaphoreType.DMA,
            pltpu.SemaphoreType.DMA,
            pltpu.SemaphoreType.DMA,
        ],
        compiler_params=pltpu.CompilerParams(
            needs_layout_passes=False,
            use_tc_tiling_on_sc=False,
            disable_bounds_checks=True,
        ),
    )
    def sc_split(in_hbm, *rest):
        outs = rest[:_F]
        in_v = rest[_F:_F + 2]
        out_v = rest[_F + 2:_F + 4]
        in_sem = rest[_F + 4:_F + 6]
        out_sem = rest[_F + 6:_F + 8]

        wid = lax.axis_index("s") * 2 + lax.axis_index("c")
        nch = ch_base + jnp.where(wid < extra, 1, 0)
        ck0 = ch_base * wid + jnp.minimum(wid, extra)
        iota_f = lax.iota(jnp.int32, _L) * _F

        def issue_in(c_rel, b):
            chunk = ck0 + c_rel
            pltpu.async_copy(
                in_hbm.at[pl.ds(chunk * (_R * _F), _R * _F)], in_v[b], in_sem[b]
            )

        def do_chunk(c_rel, b):
            chunk = ck0 + c_rel
            row0 = chunk * _R

            @pl.when(c_rel >= 2)
            def _drain_outs():
                for j in range(_F):
                    pltpu.make_async_copy(
                        out_v[b].at[j], outs[j].at[pl.ds(row0, _R)], out_sem[b]
                    ).wait()

            pltpu.make_async_copy(
                in_hbm.at[pl.ds(chunk * (_R * _F), _R * _F)], in_v[b], in_sem[b]
            ).wait()

            def r_body(r, carry):
                base_vec = iota_f + r * (_L * _F)
                for j in range(_F):
                    vals = plsc.load_gather(in_v[b], [base_vec + j])
                    out_v[b][j, pl.ds(r * _L, _L)] = vals
                return carry

            lax.fori_loop(0, _R // _L, r_body, 0, unroll=2)

            @pl.when(c_rel + 2 < nch)
            def _next_in():
                issue_in(c_rel + 2, b)

            for j in range(_F):
                pltpu.async_copy(
                    out_v[b].at[j], outs[j].at[pl.ds(row0, _R)], out_sem[b]
                )

        issue_in(0, 0)
        issue_in(1, 1)

        def loop_body(i, carry):
            do_chunk(i * 2, 0)
            do_chunk(i * 2 + 1, 1)
            return carry

        lax.fori_loop(0, nch // 2, loop_body, 0)

        @pl.when(nch % 2 == 1)
        def _tail():
            do_chunk(nch - 1, 0)

        for b in range(2):
            for j in range(_F):
                pltpu.make_async_copy(
                    out_v[b].at[j], outs[j].at[pl.ds(0, _R)], out_sem[b]
                ).wait()

    return sc_split


def kernel(inputs):
    n = inputs.shape[0]
    flat = inputs.reshape(-1)
    outs = _make_sc_kernel(n)(flat)
    return tuple(o.reshape(n, 1) for o in outs)


# speedup vs baseline: 16.0970x; 2.0026x over previous
import functools

import jax
import jax.numpy as jnp
from jax import lax
from jax.experimental import pallas as pl
from jax.experimental.pallas import tpu as pltpu
from jax.experimental.pallas import tpu_sc as plsc

_F = 22
_NW = 32
_RC = 3200
_L = 16
_GROUPS = ((0, 8), (8, 8), (16, 6))


def _make_sc_kernel(n):
    cpc = n // _RC
    kmax = (cpc + _NW - 1) // _NW
    assert kmax % 2 == 0
    mesh = plsc.VectorSubcoreMesh(core_axis_name="c", subcore_axis_name="s")

    @functools.partial(
        pl.kernel,
        mesh=mesh,
        out_type=[jax.ShapeDtypeStruct((n,), jnp.float32)] * _F,
        scratch_types=[
            pltpu.VMEM((8, _RC), jnp.float32),
            pltpu.VMEM((8, _RC), jnp.float32),
            pltpu.VMEM((8 * _RC,), jnp.float32),
            pltpu.VMEM((8 * _RC,), jnp.float32),
            pltpu.SemaphoreType.DMA,
            pltpu.SemaphoreType.DMA,
            pltpu.SemaphoreType.DMA,
            pltpu.SemaphoreType.DMA,
        ],
        compiler_params=pltpu.CompilerParams(
            needs_layout_passes=False,
            use_tc_tiling_on_sc=True,
            disable_bounds_checks=True,
        ),
    )
    def sc_split(xt_hbm, *rest):
        outs = rest[:_F]
        in_v = rest[_F:_F + 2]
        out_v = rest[_F + 2:_F + 4]
        in_sem = rest[_F + 4:_F + 6]
        out_sem = rest[_F + 6:_F + 8]
        wid = lax.axis_index("s") * 2 + lax.axis_index("c")

        for gi, (j0, gn) in enumerate(_GROUPS):
            w_g = (wid + gi * 11) % _NW
            nch = (cpc - w_g + _NW - 1) // _NW

            def issue_in(k, b, j0=j0, gn=gn, w_g=w_g):
                i0 = (w_g + k * _NW) * _RC
                pltpu.async_copy(
                    xt_hbm.at[pl.ds(j0, gn), pl.ds(i0, _RC)],
                    in_v[b].at[pl.ds(0, gn)],
                    in_sem[b],
                )

            def out_copies(k, b, j0=j0, gn=gn, w_g=w_g):
                i0 = (w_g + k * _NW) * _RC
                return [
                    (out_v[b].at[pl.ds(s * _RC, _RC)],
                     outs[j0 + s].at[pl.ds(i0, _RC)])
                    for s in range(gn)
                ]

            def do_slot(k, b, j0=j0, gn=gn, w_g=w_g, nch=nch):
                @pl.when(k >= 2)
                def _drain():
                    for src, dst in out_copies(k - 2, b):
                        pltpu.make_async_copy(src, dst, out_sem[b]).wait()

                @pl.when(k < nch)
                def _work():
                    i0 = (w_g + k * _NW) * _RC
                    pltpu.make_async_copy(
                        xt_hbm.at[pl.ds(j0, gn), pl.ds(i0, _RC)],
                        in_v[b].at[pl.ds(0, gn)],
                        in_sem[b],
                    ).wait()

                    def depad(l, carry):
                        for s in range(gn):
                            v = in_v[b][s, pl.ds(l * _L, _L)]
                            out_v[b][pl.ds(s * _RC + l * _L, _L)] = v
                        return carry

                    lax.fori_loop(0, _RC // _L, depad, 0, unroll=4)
                    for src, dst in out_copies(k, b):
                        pltpu.async_copy(src, dst, out_sem[b])

                @pl.when(k + 2 < nch)
                def _next():
                    issue_in(k + 2, b)

            issue_in(0, 0)
            issue_in(1, 1)

            def loop_body(i, carry):
                do_slot(i * 2, 0)
                do_slot(i * 2 + 1, 1)
                return carry

            lax.fori_loop(0, kmax // 2, loop_body, 0)

            for b in range(2):
                @pl.when(kmax - 2 + b < nch)
                def _final_drain(b=b):
                    for src, dst in out_copies(kmax - 2 + b, b):
                        pltpu.make_async_copy(src, dst, out_sem[b]).wait()

    return sc_split


def kernel(inputs):
    n = inputs.shape[0]
    xt = inputs.T
    outs = _make_sc_kernel(n)(xt)
    return tuple(o.reshape(n, 1) for o in outs)
